# Initial kernel scaffold; baseline (speedup 1.0000x reference)
import jax
import jax.numpy as jnp
from jax.experimental import pallas as pl


def kernel(adj, user_emb, item_emb):
    raise NotImplementedError("write your pallas kernel here")


# speedup vs baseline: 1.7600x; 1.7600x over previous
import functools

import jax
import jax.numpy as jnp
from jax.experimental import pallas as pl
from jax.experimental.pallas import tpu as pltpu

_LEAKY = 0.5


def _pass1_body(adj_ref, iemb_ref, uemb_ref, ti_ref, tu_ref):
    r = pl.program_id(0)

    @pl.when(r == 0)
    def _init():
        tu_ref[...] = jnp.zeros_like(tu_ref)

    adj = adj_ref[...]
    ti_ref[...] = jnp.dot(adj, iemb_ref[...], preferred_element_type=jnp.float32)
    tu_ref[...] += jax.lax.dot_general(
        adj, uemb_ref[...], (((0,), (0,)), ((), ())),
        preferred_element_type=jnp.float32)


def _pass2_body(adj_ref, tu_ref, ti_ref, hu_ref, hi_ref, *, nsteps):
    r = pl.program_id(0)

    @pl.when(r == 0)
    def _init():
        hi_ref[...] = jnp.zeros_like(hi_ref)

    adj = adj_ref[...]
    hu = jnp.dot(adj, tu_ref[...], preferred_element_type=jnp.float32)
    hu_ref[...] = jnp.where(hu >= 0, hu, _LEAKY * hu)
    hi_ref[...] += jax.lax.dot_general(
        adj, ti_ref[...], (((0,), (0,)), ((), ())),
        preferred_element_type=jnp.float32)

    @pl.when(r == nsteps - 1)
    def _act():
        hi = hi_ref[...]
        hi_ref[...] = jnp.where(hi >= 0, hi, _LEAKY * hi)


@functools.partial(jax.jit, static_argnames=("stripe",))
def _dhcf(adj, user_emb, item_emb, stripe=256):
    n_u, n_i = adj.shape
    d = user_emb.shape[1]
    nsteps = n_u // stripe

    grid = (nsteps,)
    params = pltpu.CompilerParams(dimension_semantics=("arbitrary",))

    t_i, t_u = pl.pallas_call(
        _pass1_body,
        grid=grid,
        in_specs=[
            pl.BlockSpec((stripe, n_i), lambda r: (r, 0)),
            pl.BlockSpec((n_i, d), lambda r: (0, 0)),
            pl.BlockSpec((stripe, d), lambda r: (r, 0)),
        ],
        out_specs=[
            pl.BlockSpec((stripe, d), lambda r: (r, 0)),
            pl.BlockSpec((n_i, d), lambda r: (0, 0)),
        ],
        out_shape=[
            jax.ShapeDtypeStruct((n_u, d), jnp.float32),
            jax.ShapeDtypeStruct((n_i, d), jnp.float32),
        ],
        compiler_params=params,
    )(adj, item_emb, user_emb)

    h_u, h_i = pl.pallas_call(
        functools.partial(_pass2_body, nsteps=nsteps),
        grid=grid,
        in_specs=[
            pl.BlockSpec((stripe, n_i), lambda r: (r, 0)),
            pl.BlockSpec((n_i, d), lambda r: (0, 0)),
            pl.BlockSpec((stripe, d), lambda r: (r, 0)),
        ],
        out_specs=[
            pl.BlockSpec((stripe, d), lambda r: (r, 0)),
            pl.BlockSpec((n_i, d), lambda r: (0, 0)),
        ],
        out_shape=[
            jax.ShapeDtypeStruct((n_u, d), jnp.float32),
            jax.ShapeDtypeStruct((n_i, d), jnp.float32),
        ],
        compiler_params=params,
    )(adj, t_u, t_i)

    user_all = jnp.concatenate([user_emb, h_u, h_u], axis=1)
    item_all = jnp.concatenate([item_emb, h_i, h_i], axis=1)
    return user_all, item_all


def kernel(adj, user_emb, item_emb):
    return _dhcf(adj, user_emb, item_emb)
